# baseline (speedup 1.0000x reference)
import jax
import jax.numpy as jnp
from jax import lax
from jax.experimental import pallas as pl
from jax.experimental.pallas import tpu as pltpu
from jax.experimental.pallas import tpu_sc as plsc

B, S, V = 256, 100, 128
ROW = V * 3
NW = 32
ITEMS_PER_W = B // NW
CH = 10
NCH = S // CH
NG = V // 16


def _sc_body(x_hbm, out_hbm, buf, out_row, m_ref):
    wid = lax.axis_index("s") * 2 + lax.axis_index("c")
    lane = lax.iota(jnp.int32, 16)
    zero16 = jnp.zeros((16,), jnp.int32)

    def per_item(i, _):
        b = wid * ITEMS_PER_W + i
        for g in range(NG):
            m_ref[pl.ds(g * 16, 16)] = zero16

        def chunk_cond(carry):
            c, gmin = carry
            return jnp.logical_and(c < NCH, gmin == 0)

        def chunk_body(carry):
            c, _ = carry
            s_top = (S - 1) - CH * c
            lo = s_top - (CH - 1)
            pltpu.sync_copy(x_hbm.at[b, pl.ds(lo, CH)], buf.at[pl.ds(lo, CH)])
            ms = []
            for g in range(NG):
                base3 = g * 48 + lane * 3
                vlane = g * 16 + lane
                cands = []
                for j in range(CH):
                    s = s_top - j
                    cls = plsc.load_gather(
                        buf, [jnp.full((16,), s, jnp.int32), vlane, zero16]
                    )
                    cands.append(jnp.where(cls > 0.0, s + 1, 0))
                while len(cands) > 1:
                    cands = [
                        jnp.maximum(cands[k], cands[k + 1])
                        for k in range(0, len(cands) - 1, 2)
                    ] + ([cands[-1]] if len(cands) % 2 else [])
                m = m_ref[pl.ds(g * 16, 16)]
                m = jnp.where(m > 0, m, cands[0])
                m_ref[pl.ds(g * 16, 16)] = m
                ms.append(m)
            while len(ms) > 1:
                ms = [
                    jnp.minimum(ms[k], ms[k + 1]) for k in range(0, len(ms) - 1, 2)
                ] + ([ms[-1]] if len(ms) % 2 else [])
            return c + 1, jnp.min(ms[0])

        lax.while_loop(chunk_cond, chunk_body, (0, 0))

        for g in range(NG):
            base3 = g * 48 + lane * 3
            vlane = g * 16 + lane
            m = m_ref[pl.ds(g * 16, 16)]
            found = m > 0
            srow = jnp.where(found, m - 1, S - 1)
            x = plsc.load_gather(buf, [srow, vlane, zero16 + 1])
            y = plsc.load_gather(buf, [srow, vlane, zero16 + 2])
            x = jnp.where(found, x, 0.0)
            y = jnp.where(found, y, 0.0)
            plsc.store_scatter(out_row, [base3], jnp.ones((16,), jnp.float32))
            plsc.store_scatter(out_row, [base3 + 1], x)
            plsc.store_scatter(out_row, [base3 + 2], y)

        pltpu.sync_copy(out_row, out_hbm.at[b])
        return 0

    lax.fori_loop(0, ITEMS_PER_W, per_item, 0)


def kernel(batch):
    mesh = plsc.VectorSubcoreMesh(core_axis_name="c", subcore_axis_name="s")
    k = pl.kernel(
        _sc_body,
        out_type=jax.ShapeDtypeStruct((B, ROW), jnp.float32),
        mesh=mesh,
        scratch_types=[
            pltpu.VMEM((S, V, 3), jnp.float32),
            pltpu.VMEM((ROW,), jnp.float32),
            pltpu.VMEM((V,), jnp.int32),
        ],
        compiler_params=pltpu.CompilerParams(
            needs_layout_passes=False, use_tc_tiling_on_sc=False
        ),
    )
    out = k(batch)
    return out.reshape(B, V, 3)


# speedup vs baseline: 397.4331x; 397.4331x over previous
import jax
import jax.numpy as jnp
from jax import lax
from jax.experimental import pallas as pl
from jax.experimental.pallas import tpu as pltpu
from jax.experimental.pallas import tpu_sc as plsc

B, S, V = 256, 100, 128
NW = 32
IPW = B // NW
CH = 10
NCH = S // CH
NG = V // 16


def _sc_body(x_hbm, out_hbm, bufc, bufx, bufy, outx, outy, outo, m_ref, sem):
    wid = lax.axis_index("s") * 2 + lax.axis_index("c")
    b0 = wid * IPW
    lane = lax.iota(jnp.int32, 16)
    zero16 = jnp.zeros((16,), jnp.int32)
    one16 = jnp.ones((16,), jnp.float32)

    def per_item(i, _):
        b = b0 + i
        for g in range(NG):
            m_ref[pl.ds(g * 16, 16)] = zero16

        def chunk_cond(carry):
            c, gmin = carry
            return jnp.logical_and(c < NCH, gmin == 0)

        def chunk_body(carry):
            c, _ = carry
            s_top = (S - 1) - CH * c
            lo = s_top - (CH - 1)
            cpy_c = pltpu.make_async_copy(
                x_hbm.at[pl.ds(lo, CH), 0, b], bufc.at[pl.ds(lo, CH)], sem
            )
            cpy_x = pltpu.make_async_copy(
                x_hbm.at[pl.ds(lo, CH), 1, b], bufx.at[pl.ds(lo, CH)], sem
            )
            cpy_y = pltpu.make_async_copy(
                x_hbm.at[pl.ds(lo, CH), 2, b], bufy.at[pl.ds(lo, CH)], sem
            )
            cpy_c.start()
            cpy_x.start()
            cpy_y.start()
            cpy_c.wait()
            cpy_x.wait()
            cpy_y.wait()
            ms = []
            for g in range(NG):
                cands = []
                for j in range(CH):
                    s = s_top - j
                    cls = bufc[s, pl.ds(g * 16, 16)]
                    cands.append(jnp.where(cls > 0.0, s + 1, 0))
                while len(cands) > 1:
                    cands = [
                        jnp.maximum(cands[k], cands[k + 1])
                        for k in range(0, len(cands) - 1, 2)
                    ] + ([cands[-1]] if len(cands) % 2 else [])
                m = m_ref[pl.ds(g * 16, 16)]
                m = jnp.where(m > 0, m, cands[0])
                m_ref[pl.ds(g * 16, 16)] = m
                ms.append(m)
            while len(ms) > 1:
                ms = [
                    jnp.minimum(ms[k], ms[k + 1]) for k in range(0, len(ms) - 1, 2)
                ] + ([ms[-1]] if len(ms) % 2 else [])
            return c + 1, jnp.min(ms[0])

        lax.while_loop(chunk_cond, chunk_body, (0, 0))

        for g in range(NG):
            vlane = g * 16 + lane
            m = m_ref[pl.ds(g * 16, 16)]
            found = m > 0
            srow = jnp.where(found, m - 1, S - 1)
            x = plsc.load_gather(bufx, [srow, vlane])
            y = plsc.load_gather(bufy, [srow, vlane])
            outx[i, pl.ds(g * 16, 16)] = jnp.where(found, x, 0.0)
            outy[i, pl.ds(g * 16, 16)] = jnp.where(found, y, 0.0)
            outo[i, pl.ds(g * 16, 16)] = one16
        return 0

    lax.fori_loop(0, IPW, per_item, 0)
    pltpu.sync_copy(outo, out_hbm.at[0, pl.ds(b0, IPW)])
    pltpu.sync_copy(outx, out_hbm.at[1, pl.ds(b0, IPW)])
    pltpu.sync_copy(outy, out_hbm.at[2, pl.ds(b0, IPW)])


def kernel(batch):
    xt = jnp.transpose(batch, (1, 3, 0, 2))
    mesh = plsc.VectorSubcoreMesh(core_axis_name="c", subcore_axis_name="s")
    k = pl.kernel(
        _sc_body,
        out_type=jax.ShapeDtypeStruct((3, B, V), jnp.float32),
        mesh=mesh,
        scratch_types=[
            pltpu.VMEM((S, V), jnp.float32),
            pltpu.VMEM((S, V), jnp.float32),
            pltpu.VMEM((S, V), jnp.float32),
            pltpu.VMEM((IPW, V), jnp.float32),
            pltpu.VMEM((IPW, V), jnp.float32),
            pltpu.VMEM((IPW, V), jnp.float32),
            pltpu.VMEM((V,), jnp.int32),
            pltpu.SemaphoreType.DMA,
        ],
        compiler_params=pltpu.CompilerParams(
            needs_layout_passes=False, use_tc_tiling_on_sc=False
        ),
    )
    out = k(xt)
    return jnp.transpose(out, (1, 2, 0))


# speedup vs baseline: 433.4023x; 1.0905x over previous
import jax
import jax.numpy as jnp
from jax import lax
from jax.experimental import pallas as pl
from jax.experimental.pallas import tpu as pltpu
from jax.experimental.pallas import tpu_sc as plsc

B, S, V = 256, 100, 128
NW = 32
IPW = B // NW
CH = 10
NCH = S // CH
NG = V // 16
LO0 = S - CH


def _maxtree(vals):
    vals = list(vals)
    while len(vals) > 1:
        vals = [
            jnp.maximum(vals[k], vals[k + 1]) for k in range(0, len(vals) - 1, 2)
        ] + ([vals[-1]] if len(vals) % 2 else [])
    return vals[0]


def _mintree(vals):
    vals = list(vals)
    while len(vals) > 1:
        vals = [
            jnp.minimum(vals[k], vals[k + 1]) for k in range(0, len(vals) - 1, 2)
        ] + ([vals[-1]] if len(vals) % 2 else [])
    return vals[0]


def _sc_body(x_hbm, out_hbm, bufc, bufx, bufy, outx, outy, outo, m_big, sems, sem2):
    wid = lax.axis_index("s") * 2 + lax.axis_index("c")
    b0 = wid * IPW
    lane = lax.iota(jnp.int32, 16)
    one16 = jnp.ones((16,), jnp.float32)

    def _chunk0_copies(i):
        b = b0 + i
        return [
            pltpu.make_async_copy(
                x_hbm.at[pl.ds(LO0, CH), ch, b], buf.at[i], sems.at[i]
            )
            for ch, buf in ((0, bufc), (1, bufx), (2, bufy))
        ]

    for i in range(IPW):
        for cpy in _chunk0_copies(i):
            cpy.start()

    item_mins = []
    for i in range(IPW):
        for cpy in _chunk0_copies(i):
            cpy.wait()
        i_splat = jnp.full((16,), i, jnp.int32)
        ms = []
        for g in range(NG):
            vlane = g * 16 + lane
            cands = [
                jnp.where(bufc[i, r, pl.ds(g * 16, 16)] > 0.0, LO0 + r + 1, 0)
                for r in range(CH)
            ]
            m = _maxtree(cands)
            found = m > 0
            rrow = jnp.where(found, m - 1 - LO0, 0)
            x = plsc.load_gather(bufx, [i_splat, rrow, vlane])
            y = plsc.load_gather(bufy, [i_splat, rrow, vlane])
            outx[i, pl.ds(g * 16, 16)] = jnp.where(found, x, 0.0)
            outy[i, pl.ds(g * 16, 16)] = jnp.where(found, y, 0.0)
            outo[i, pl.ds(g * 16, 16)] = one16
            m_big[pl.ds(i * V + g * 16, 16)] = m
            ms.append(m)
        item_mins.append(jnp.min(_mintree(ms)))

    gmin = item_mins[0]
    for v in item_mins[1:]:
        gmin = jnp.minimum(gmin, v)

    @pl.when(gmin == 0)
    def _pass2():
        def per_item(i, _):
            b = b0 + i
            m8 = [m_big[pl.ds(i * V + g * 16, 16)] for g in range(NG)]
            imin = jnp.min(_mintree(m8))

            @pl.when(imin == 0)
            def _scan_older():
                def cond(carry):
                    c, cmin = carry
                    return jnp.logical_and(c < NCH, cmin == 0)

                def body(carry):
                    c, _ = carry
                    lo = S - CH * (c + 1)
                    for ch, buf in ((0, bufc), (1, bufx), (2, bufy)):
                        pltpu.make_async_copy(
                            x_hbm.at[pl.ds(lo, CH), ch, b], buf.at[i], sem2
                        ).start()
                    for ch, buf in ((0, bufc), (1, bufx), (2, bufy)):
                        pltpu.make_async_copy(
                            x_hbm.at[pl.ds(lo, CH), ch, b], buf.at[i], sem2
                        ).wait()
                    i_splat = jnp.full((16,), i, jnp.int32)
                    ms = []
                    for g in range(NG):
                        vlane = g * 16 + lane
                        cands = [
                            jnp.where(
                                bufc[i, r, pl.ds(g * 16, 16)] > 0.0, lo + r + 1, 0
                            )
                            for r in range(CH)
                        ]
                        mc = _maxtree(cands)
                        mo = m_big[pl.ds(i * V + g * 16, 16)]
                        newly = jnp.logical_and(mo == 0, mc > 0)
                        rrow = jnp.where(newly, mc - 1 - lo, 0)
                        x = plsc.load_gather(bufx, [i_splat, rrow, vlane])
                        y = plsc.load_gather(bufy, [i_splat, rrow, vlane])
                        xo = outx[i, pl.ds(g * 16, 16)]
                        yo = outy[i, pl.ds(g * 16, 16)]
                        outx[i, pl.ds(g * 16, 16)] = jnp.where(newly, x, xo)
                        outy[i, pl.ds(g * 16, 16)] = jnp.where(newly, y, yo)
                        mn = jnp.where(mo > 0, mo, mc)
                        m_big[pl.ds(i * V + g * 16, 16)] = mn
                        ms.append(mn)
                    return c + 1, jnp.min(_mintree(ms))

                lax.while_loop(cond, body, (1, 0))

            return 0

        lax.fori_loop(0, IPW, per_item, 0)

    pltpu.sync_copy(outo, out_hbm.at[0, pl.ds(b0, IPW)])
    pltpu.sync_copy(outx, out_hbm.at[1, pl.ds(b0, IPW)])
    pltpu.sync_copy(outy, out_hbm.at[2, pl.ds(b0, IPW)])


def kernel(batch):
    xt = jnp.transpose(batch, (1, 3, 0, 2))
    mesh = plsc.VectorSubcoreMesh(core_axis_name="c", subcore_axis_name="s")
    k = pl.kernel(
        _sc_body,
        out_type=jax.ShapeDtypeStruct((3, B, V), jnp.float32),
        mesh=mesh,
        scratch_types=[
            pltpu.VMEM((IPW, CH, V), jnp.float32),
            pltpu.VMEM((IPW, CH, V), jnp.float32),
            pltpu.VMEM((IPW, CH, V), jnp.float32),
            pltpu.VMEM((IPW, V), jnp.float32),
            pltpu.VMEM((IPW, V), jnp.float32),
            pltpu.VMEM((IPW, V), jnp.float32),
            pltpu.VMEM((IPW * V,), jnp.int32),
            pltpu.SemaphoreType.DMA((IPW,)),
            pltpu.SemaphoreType.DMA,
        ],
        compiler_params=pltpu.CompilerParams(
            needs_layout_passes=False, use_tc_tiling_on_sc=False
        ),
    )
    out = k(xt)
    return jnp.transpose(out, (1, 2, 0))


# speedup vs baseline: 434.5066x; 1.0025x over previous
import jax
import jax.numpy as jnp
from jax import lax
from jax.experimental import pallas as pl
from jax.experimental.pallas import tpu as pltpu
from jax.experimental.pallas import tpu_sc as plsc

B, S, V = 256, 100, 128
NW = 32
IPW = B // NW
CH = 10
NCH = S // CH
NG = V // 16
LO0 = S - CH


def _maxtree(vals):
    vals = list(vals)
    while len(vals) > 1:
        vals = [
            jnp.maximum(vals[k], vals[k + 1]) for k in range(0, len(vals) - 1, 2)
        ] + ([vals[-1]] if len(vals) % 2 else [])
    return vals[0]


def _mintree(vals):
    vals = list(vals)
    while len(vals) > 1:
        vals = [
            jnp.minimum(vals[k], vals[k + 1]) for k in range(0, len(vals) - 1, 2)
        ] + ([vals[-1]] if len(vals) % 2 else [])
    return vals[0]


def _sc_body(x_hbm, out_hbm, bufc, bufx, bufy, outx, outy, outo, m_big, sems, sem2):
    wid = lax.axis_index("s") * 2 + lax.axis_index("c")
    b0 = wid * IPW
    lane = lax.iota(jnp.int32, 16)
    one16 = jnp.ones((16,), jnp.float32)

    def _chunk0_copies(i):
        b = b0 + i
        return [
            pltpu.make_async_copy(
                x_hbm.at[pl.ds(LO0, CH), ch, b], buf.at[i], sems.at[i]
            )
            for ch, buf in ((0, bufc), (1, bufx), (2, bufy))
        ]

    for i in range(IPW):
        for cpy in _chunk0_copies(i):
            cpy.start()

    item_mins = []
    for i in range(IPW):
        for cpy in _chunk0_copies(i):
            cpy.wait()
        i_splat = jnp.full((16,), i, jnp.int32)
        ms = []
        for g in range(NG):
            vlane = g * 16 + lane
            cands = [
                jnp.where(bufc[i, r, pl.ds(g * 16, 16)] > 0.0, LO0 + r + 1, 0)
                for r in range(CH)
            ]
            m = _maxtree(cands)
            found = m > 0
            rrow = jnp.where(found, m - 1 - LO0, 0)
            x = plsc.load_gather(bufx, [i_splat, rrow, vlane])
            y = plsc.load_gather(bufy, [i_splat, rrow, vlane])
            outx[i, pl.ds(g * 16, 16)] = jnp.where(found, x, 0.0)
            outy[i, pl.ds(g * 16, 16)] = jnp.where(found, y, 0.0)
            outo[i, pl.ds(g * 16, 16)] = one16
            m_big[pl.ds(i * V + g * 16, 16)] = m
            ms.append(m)
        item_mins.append(jnp.min(_mintree(ms)))

    gmin = item_mins[0]
    for v in item_mins[1:]:
        gmin = jnp.minimum(gmin, v)

    @pl.when(gmin == 0)
    def _pass2():
        def per_item(i, _):
            b = b0 + i
            m8 = [m_big[pl.ds(i * V + g * 16, 16)] for g in range(NG)]
            imin = jnp.min(_mintree(m8))

            @pl.when(imin == 0)
            def _scan_older():
                def cond(carry):
                    c, cmin = carry
                    return jnp.logical_and(c < NCH, cmin == 0)

                def body(carry):
                    c, _ = carry
                    lo = S - CH * (c + 1)
                    for ch, buf in ((0, bufc), (1, bufx), (2, bufy)):
                        pltpu.make_async_copy(
                            x_hbm.at[pl.ds(lo, CH), ch, b], buf.at[i], sem2
                        ).start()
                    for ch, buf in ((0, bufc), (1, bufx), (2, bufy)):
                        pltpu.make_async_copy(
                            x_hbm.at[pl.ds(lo, CH), ch, b], buf.at[i], sem2
                        ).wait()
                    i_splat = jnp.full((16,), i, jnp.int32)
                    ms = []
                    for g in range(NG):
                        vlane = g * 16 + lane
                        cands = [
                            jnp.where(
                                bufc[i, r, pl.ds(g * 16, 16)] > 0.0, lo + r + 1, 0
                            )
                            for r in range(CH)
                        ]
                        mc = _maxtree(cands)
                        mo = m_big[pl.ds(i * V + g * 16, 16)]
                        newly = jnp.logical_and(mo == 0, mc > 0)
                        rrow = jnp.where(newly, mc - 1 - lo, 0)
                        x = plsc.load_gather(bufx, [i_splat, rrow, vlane])
                        y = plsc.load_gather(bufy, [i_splat, rrow, vlane])
                        xo = outx[i, pl.ds(g * 16, 16)]
                        yo = outy[i, pl.ds(g * 16, 16)]
                        outx[i, pl.ds(g * 16, 16)] = jnp.where(newly, x, xo)
                        outy[i, pl.ds(g * 16, 16)] = jnp.where(newly, y, yo)
                        mn = jnp.where(mo > 0, mo, mc)
                        m_big[pl.ds(i * V + g * 16, 16)] = mn
                        ms.append(mn)
                    return c + 1, jnp.min(_mintree(ms))

                lax.while_loop(cond, body, (1, 0))

            return 0

        lax.fori_loop(0, IPW, per_item, 0)

    pltpu.sync_copy(outo, out_hbm.at[0, pl.ds(b0, IPW)])
    pltpu.sync_copy(outx, out_hbm.at[1, pl.ds(b0, IPW)])
    pltpu.sync_copy(outy, out_hbm.at[2, pl.ds(b0, IPW)])


def kernel(batch):
    xt = jnp.transpose(batch, (1, 3, 0, 2))
    mesh = plsc.VectorSubcoreMesh(core_axis_name="c", subcore_axis_name="s")
    k = pl.kernel(
        _sc_body,
        out_type=jax.ShapeDtypeStruct((3, B, V), jnp.float32),
        mesh=mesh,
        scratch_types=[
            pltpu.VMEM((IPW, CH, V), jnp.float32),
            pltpu.VMEM((IPW, CH, V), jnp.float32),
            pltpu.VMEM((IPW, CH, V), jnp.float32),
            pltpu.VMEM((IPW, V), jnp.float32),
            pltpu.VMEM((IPW, V), jnp.float32),
            pltpu.VMEM((IPW, V), jnp.float32),
            pltpu.VMEM((IPW * V,), jnp.int32),
            pltpu.SemaphoreType.DMA((IPW,)),
            pltpu.SemaphoreType.DMA,
        ],
        compiler_params=pltpu.CompilerParams(
            needs_layout_passes=False,
            use_tc_tiling_on_sc=False,
            skip_device_barrier=True,
        ),
    )
    out = k(xt)
    return jnp.transpose(out, (1, 2, 0))


# speedup vs baseline: 435.0296x; 1.0012x over previous
import jax
import jax.numpy as jnp
from jax import lax
from jax.experimental import pallas as pl
from jax.experimental.pallas import tpu as pltpu
from jax.experimental.pallas import tpu_sc as plsc

B, S, V = 256, 100, 128
NW = 32
IPW = B // NW
CH = 10
NCH = S // CH
NG = V // 16
LO0 = S - CH


def _maxtree(vals):
    vals = list(vals)
    while len(vals) > 1:
        vals = [
            jnp.maximum(vals[k], vals[k + 1]) for k in range(0, len(vals) - 1, 2)
        ] + ([vals[-1]] if len(vals) % 2 else [])
    return vals[0]


def _mintree(vals):
    vals = list(vals)
    while len(vals) > 1:
        vals = [
            jnp.minimum(vals[k], vals[k + 1]) for k in range(0, len(vals) - 1, 2)
        ] + ([vals[-1]] if len(vals) % 2 else [])
    return vals[0]


def _sc_body(x_hbm, out_hbm, bufc, bufx, bufy, outx, outy, outo, m_big, sems, sem2):
    wid = lax.axis_index("s") * 2 + lax.axis_index("c")
    b0 = wid * IPW
    lane = lax.iota(jnp.int32, 16)
    one16 = jnp.ones((16,), jnp.float32)

    def _chunk0_copies(i):
        b = b0 + i
        return [
            pltpu.make_async_copy(
                x_hbm.at[pl.ds(LO0, CH), ch, b], buf.at[i], sems.at[i]
            )
            for ch, buf in ((0, bufc), (1, bufx), (2, bufy))
        ]

    for i in range(IPW):
        for cpy in _chunk0_copies(i):
            cpy.start()

    item_mins = []
    for i in range(IPW):
        for cpy in _chunk0_copies(i):
            cpy.wait()
        i_splat = jnp.full((16,), i, jnp.int32)
        ms = []
        for g in range(NG):
            vlane = g * 16 + lane
            cands = [
                jnp.where(bufc[i, r, pl.ds(g * 16, 16)] > 0.0, LO0 + r + 1, 0)
                for r in range(CH)
            ]
            m = _maxtree(cands)
            found = m > 0
            rrow = jnp.where(found, m - 1 - LO0, 0)
            x = plsc.load_gather(bufx, [i_splat, rrow, vlane])
            y = plsc.load_gather(bufy, [i_splat, rrow, vlane])
            outx[i, pl.ds(g * 16, 16)] = jnp.where(found, x, 0.0)
            outy[i, pl.ds(g * 16, 16)] = jnp.where(found, y, 0.0)
            outo[i, pl.ds(g * 16, 16)] = one16
            m_big[pl.ds(i * V + g * 16, 16)] = m
            ms.append(m)
        item_mins.append(jnp.min(_mintree(ms)))

    gmin = item_mins[0]
    for v in item_mins[1:]:
        gmin = jnp.minimum(gmin, v)

    @pl.when(gmin == 0)
    def _pass2():
        def per_item(i, _):
            b = b0 + i
            m8 = [m_big[pl.ds(i * V + g * 16, 16)] for g in range(NG)]
            imin = jnp.min(_mintree(m8))

            @pl.when(imin == 0)
            def _scan_older():
                def cond(carry):
                    c, cmin = carry
                    return jnp.logical_and(c < NCH, cmin == 0)

                def body(carry):
                    c, _ = carry
                    lo = S - CH * (c + 1)
                    for ch, buf in ((0, bufc), (1, bufx), (2, bufy)):
                        pltpu.make_async_copy(
                            x_hbm.at[pl.ds(lo, CH), ch, b], buf.at[i], sem2
                        ).start()
                    for ch, buf in ((0, bufc), (1, bufx), (2, bufy)):
                        pltpu.make_async_copy(
                            x_hbm.at[pl.ds(lo, CH), ch, b], buf.at[i], sem2
                        ).wait()
                    i_splat = jnp.full((16,), i, jnp.int32)
                    ms = []
                    for g in range(NG):
                        vlane = g * 16 + lane
                        cands = [
                            jnp.where(
                                bufc[i, r, pl.ds(g * 16, 16)] > 0.0, lo + r + 1, 0
                            )
                            for r in range(CH)
                        ]
                        mc = _maxtree(cands)
                        mo = m_big[pl.ds(i * V + g * 16, 16)]
                        newly = jnp.logical_and(mo == 0, mc > 0)
                        rrow = jnp.where(newly, mc - 1 - lo, 0)
                        x = plsc.load_gather(bufx, [i_splat, rrow, vlane])
                        y = plsc.load_gather(bufy, [i_splat, rrow, vlane])
                        xo = outx[i, pl.ds(g * 16, 16)]
                        yo = outy[i, pl.ds(g * 16, 16)]
                        outx[i, pl.ds(g * 16, 16)] = jnp.where(newly, x, xo)
                        outy[i, pl.ds(g * 16, 16)] = jnp.where(newly, y, yo)
                        mn = jnp.where(mo > 0, mo, mc)
                        m_big[pl.ds(i * V + g * 16, 16)] = mn
                        ms.append(mn)
                    return c + 1, jnp.min(_mintree(ms))

                lax.while_loop(cond, body, (1, 0))

            return 0

        lax.fori_loop(0, IPW, per_item, 0)

    pltpu.sync_copy(outo, out_hbm.at[0, pl.ds(b0, IPW)])
    pltpu.sync_copy(outx, out_hbm.at[1, pl.ds(b0, IPW)])
    pltpu.sync_copy(outy, out_hbm.at[2, pl.ds(b0, IPW)])


def kernel(batch):
    xt = jnp.transpose(batch, (1, 3, 0, 2))
    mesh = plsc.VectorSubcoreMesh(core_axis_name="c", subcore_axis_name="s")
    k = pl.kernel(
        _sc_body,
        out_type=jax.ShapeDtypeStruct((3, B, V), jnp.float32),
        mesh=mesh,
        scratch_types=[
            pltpu.VMEM((IPW, CH, V), jnp.float32),
            pltpu.VMEM((IPW, CH, V), jnp.float32),
            pltpu.VMEM((IPW, CH, V), jnp.float32),
            pltpu.VMEM((IPW, V), jnp.float32),
            pltpu.VMEM((IPW, V), jnp.float32),
            pltpu.VMEM((IPW, V), jnp.float32),
            pltpu.VMEM((IPW * V,), jnp.int32),
            pltpu.SemaphoreType.DMA((IPW,)),
            pltpu.SemaphoreType.DMA,
        ],
        compiler_params=pltpu.CompilerParams(
            needs_layout_passes=False, use_tc_tiling_on_sc=False
        ),
    )
    out = k(xt)
    return jnp.transpose(out, (1, 2, 0))


# speedup vs baseline: 457.8254x; 1.0524x over previous
import jax
import jax.numpy as jnp
from jax import lax
from jax.experimental import pallas as pl
from jax.experimental.pallas import tpu as pltpu
from jax.experimental.pallas import tpu_sc as plsc

B, S, V = 256, 100, 128
NW = 32
IPW = B // NW
CH = 10
NCH = S // CH
NG = V // 16
LO0 = S - CH


def _maxtree(vals):
    vals = list(vals)
    while len(vals) > 1:
        vals = [
            jnp.maximum(vals[k], vals[k + 1]) for k in range(0, len(vals) - 1, 2)
        ] + ([vals[-1]] if len(vals) % 2 else [])
    return vals[0]


def _mintree(vals):
    vals = list(vals)
    while len(vals) > 1:
        vals = [
            jnp.minimum(vals[k], vals[k + 1]) for k in range(0, len(vals) - 1, 2)
        ] + ([vals[-1]] if len(vals) % 2 else [])
    return vals[0]


def _sc_body(x_hbm, out_hbm, bufc, bufx, bufy, outx, outy, outo, m_big, sems, sem2):
    wid = lax.axis_index("s") * 2 + lax.axis_index("c")
    b0 = wid * IPW
    lane = lax.iota(jnp.int32, 16)
    one16 = jnp.ones((16,), jnp.float32)

    def _chunk0_copies(i):
        b = b0 + i
        return [
            pltpu.make_async_copy(
                x_hbm.at[pl.ds(LO0, CH), ch, b], buf.at[i], sems.at[i]
            )
            for ch, buf in ((0, bufc), (1, bufx), (2, bufy))
        ]

    for i in range(IPW):
        for cpy in _chunk0_copies(i):
            cpy.start()

    def pass1_item(i, gmin_acc):
        for cpy in _chunk0_copies(i):
            cpy.wait()
        i_splat = jnp.full((16,), i, jnp.int32)
        ms = []
        for g in range(NG):
            vlane = g * 16 + lane
            cands = [
                jnp.where(bufc[i, r, pl.ds(g * 16, 16)] > 0.0, LO0 + r + 1, 0)
                for r in range(CH)
            ]
            m = _maxtree(cands)
            found = m > 0
            rrow = jnp.where(found, m - 1 - LO0, 0)
            x = plsc.load_gather(bufx, [i_splat, rrow, vlane])
            y = plsc.load_gather(bufy, [i_splat, rrow, vlane])
            outx[i, pl.ds(g * 16, 16)] = jnp.where(found, x, 0.0)
            outy[i, pl.ds(g * 16, 16)] = jnp.where(found, y, 0.0)
            outo[i, pl.ds(g * 16, 16)] = one16
            m_big[pl.ds(i * V + g * 16, 16)] = m
            ms.append(m)
        return jnp.minimum(gmin_acc, jnp.min(_mintree(ms)))

    gmin = lax.fori_loop(0, IPW, pass1_item, jnp.int32(2**30))

    @pl.when(gmin == 0)
    def _pass2():
        def per_item(i, _):
            b = b0 + i
            m8 = [m_big[pl.ds(i * V + g * 16, 16)] for g in range(NG)]
            imin = jnp.min(_mintree(m8))

            @pl.when(imin == 0)
            def _scan_older():
                def cond(carry):
                    c, cmin = carry
                    return jnp.logical_and(c < NCH, cmin == 0)

                def body(carry):
                    c, _ = carry
                    lo = S - CH * (c + 1)
                    for ch, buf in ((0, bufc), (1, bufx), (2, bufy)):
                        pltpu.make_async_copy(
                            x_hbm.at[pl.ds(lo, CH), ch, b], buf.at[i], sem2
                        ).start()
                    for ch, buf in ((0, bufc), (1, bufx), (2, bufy)):
                        pltpu.make_async_copy(
                            x_hbm.at[pl.ds(lo, CH), ch, b], buf.at[i], sem2
                        ).wait()
                    i_splat = jnp.full((16,), i, jnp.int32)
                    ms = []
                    for g in range(NG):
                        vlane = g * 16 + lane
                        cands = [
                            jnp.where(
                                bufc[i, r, pl.ds(g * 16, 16)] > 0.0, lo + r + 1, 0
                            )
                            for r in range(CH)
                        ]
                        mc = _maxtree(cands)
                        mo = m_big[pl.ds(i * V + g * 16, 16)]
                        newly = jnp.logical_and(mo == 0, mc > 0)
                        rrow = jnp.where(newly, mc - 1 - lo, 0)
                        x = plsc.load_gather(bufx, [i_splat, rrow, vlane])
                        y = plsc.load_gather(bufy, [i_splat, rrow, vlane])
                        xo = outx[i, pl.ds(g * 16, 16)]
                        yo = outy[i, pl.ds(g * 16, 16)]
                        outx[i, pl.ds(g * 16, 16)] = jnp.where(newly, x, xo)
                        outy[i, pl.ds(g * 16, 16)] = jnp.where(newly, y, yo)
                        mn = jnp.where(mo > 0, mo, mc)
                        m_big[pl.ds(i * V + g * 16, 16)] = mn
                        ms.append(mn)
                    return c + 1, jnp.min(_mintree(ms))

                lax.while_loop(cond, body, (1, 0))

            return 0

        lax.fori_loop(0, IPW, per_item, 0)

    pltpu.sync_copy(outo, out_hbm.at[0, pl.ds(b0, IPW)])
    pltpu.sync_copy(outx, out_hbm.at[1, pl.ds(b0, IPW)])
    pltpu.sync_copy(outy, out_hbm.at[2, pl.ds(b0, IPW)])


def kernel(batch):
    xt = jnp.transpose(batch, (1, 3, 0, 2))
    mesh = plsc.VectorSubcoreMesh(core_axis_name="c", subcore_axis_name="s")
    k = pl.kernel(
        _sc_body,
        out_type=jax.ShapeDtypeStruct((3, B, V), jnp.float32),
        mesh=mesh,
        scratch_types=[
            pltpu.VMEM((IPW, CH, V), jnp.float32),
            pltpu.VMEM((IPW, CH, V), jnp.float32),
            pltpu.VMEM((IPW, CH, V), jnp.float32),
            pltpu.VMEM((IPW, V), jnp.float32),
            pltpu.VMEM((IPW, V), jnp.float32),
            pltpu.VMEM((IPW, V), jnp.float32),
            pltpu.VMEM((IPW * V,), jnp.int32),
            pltpu.SemaphoreType.DMA((IPW,)),
            pltpu.SemaphoreType.DMA,
        ],
        compiler_params=pltpu.CompilerParams(
            needs_layout_passes=False, use_tc_tiling_on_sc=False
        ),
    )
    out = k(xt)
    return jnp.transpose(out, (1, 2, 0))


# speedup vs baseline: 480.0620x; 1.0486x over previous
import jax
import jax.numpy as jnp
from jax import lax
from jax.experimental import pallas as pl
from jax.experimental.pallas import tpu as pltpu
from jax.experimental.pallas import tpu_sc as plsc

B, S, V = 256, 100, 128
NW = 32
IPW = B // NW
CH = 10
NCH = S // CH
NG = V // 16
LO0 = S - CH


def _maxtree(vals):
    vals = list(vals)
    while len(vals) > 1:
        vals = [
            jnp.maximum(vals[k], vals[k + 1]) for k in range(0, len(vals) - 1, 2)
        ] + ([vals[-1]] if len(vals) % 2 else [])
    return vals[0]


def _sc_body(x_hbm, out_hbm, bufc, bufx, bufy, outx, outy, outo, m_ref, sems, sem2):
    wid = lax.axis_index("s") * 2 + lax.axis_index("c")
    b0 = wid * IPW
    lane = lax.iota(jnp.int32, 16)
    one16 = jnp.ones((16,), jnp.float32)
    zero16f = jnp.zeros((16,), jnp.float32)

    def _copies(i, lo, sem):
        b = b0 + i
        return [
            pltpu.make_async_copy(
                x_hbm.at[pl.ds(lo, CH), ch, buf_i[0]], buf_i[1].at[i], sem
            )
            for ch, buf_i in ((0, (b, bufc)), (1, (b, bufx)), (2, (b, bufy)))
        ]

    def prefetch(i, _):
        for cpy in _copies(i, LO0, sems.at[i]):
            cpy.start()
        return 0

    lax.fori_loop(0, IPW, prefetch, 0)

    def per_item(i, _):
        i_splat = jnp.full((16,), i, jnp.int32)

        def cond(carry):
            c, cmin = carry
            return jnp.logical_or(
                c == 0, jnp.logical_and(c < NCH, cmin == 0)
            )

        def chunk(carry):
            c, _ = carry
            lo = S - CH * (c + 1)
            first = c == 0

            @pl.when(first)
            def _wait0():
                for cpy in _copies(i, LO0, sems.at[i]):
                    cpy.wait()

            @pl.when(jnp.logical_not(first))
            def _fetch_older():
                for cpy in _copies(i, lo, sem2):
                    cpy.start()
                for cpy in _copies(i, lo, sem2):
                    cpy.wait()

            def per_group(g, cmin_acc):
                gl = g * 16
                vlane = gl + lane
                cands = [
                    jnp.where(bufc[i, r, pl.ds(gl, 16)] > 0.0, lo + r + 1, 0)
                    for r in range(CH)
                ]
                mc = _maxtree(cands)
                mo = jnp.where(first, 0, m_ref[pl.ds(gl, 16)])
                newly = jnp.logical_and(mo == 0, mc > 0)
                rrow = jnp.where(newly, mc - 1 - lo, 0)
                x = plsc.load_gather(bufx, [i_splat, rrow, vlane])
                y = plsc.load_gather(bufy, [i_splat, rrow, vlane])
                xo = jnp.where(first, zero16f, outx[i, pl.ds(gl, 16)])
                yo = jnp.where(first, zero16f, outy[i, pl.ds(gl, 16)])
                outx[i, pl.ds(gl, 16)] = jnp.where(newly, x, xo)
                outy[i, pl.ds(gl, 16)] = jnp.where(newly, y, yo)
                outo[i, pl.ds(gl, 16)] = one16
                mn = jnp.where(mo > 0, mo, mc)
                m_ref[pl.ds(gl, 16)] = mn
                return jnp.minimum(cmin_acc, jnp.min(mn))

            cmin = lax.fori_loop(0, NG, per_group, jnp.int32(2**30))
            return c + 1, cmin

        lax.while_loop(cond, chunk, (0, 0))
        return 0

    lax.fori_loop(0, IPW, per_item, 0)
    pltpu.sync_copy(outo, out_hbm.at[0, pl.ds(b0, IPW)])
    pltpu.sync_copy(outx, out_hbm.at[1, pl.ds(b0, IPW)])
    pltpu.sync_copy(outy, out_hbm.at[2, pl.ds(b0, IPW)])


def kernel(batch):
    xt = jnp.transpose(batch, (1, 3, 0, 2))
    mesh = plsc.VectorSubcoreMesh(core_axis_name="c", subcore_axis_name="s")
    k = pl.kernel(
        _sc_body,
        out_type=jax.ShapeDtypeStruct((3, B, V), jnp.float32),
        mesh=mesh,
        scratch_types=[
            pltpu.VMEM((IPW, CH, V), jnp.float32),
            pltpu.VMEM((IPW, CH, V), jnp.float32),
            pltpu.VMEM((IPW, CH, V), jnp.float32),
            pltpu.VMEM((IPW, V), jnp.float32),
            pltpu.VMEM((IPW, V), jnp.float32),
            pltpu.VMEM((IPW, V), jnp.float32),
            pltpu.VMEM((V,), jnp.int32),
            pltpu.SemaphoreType.DMA((IPW,)),
            pltpu.SemaphoreType.DMA,
        ],
        compiler_params=pltpu.CompilerParams(
            needs_layout_passes=False, use_tc_tiling_on_sc=False
        ),
    )
    out = k(xt)
    return jnp.transpose(out, (1, 2, 0))


# speedup vs baseline: 485.8838x; 1.0121x over previous
import jax
import jax.numpy as jnp
from jax import lax
from jax.experimental import pallas as pl
from jax.experimental.pallas import tpu as pltpu
from jax.experimental.pallas import tpu_sc as plsc

B, S, V = 256, 100, 128
NW = 32
IPW = B // NW
CH = 10
NCH = S // CH
NG = V // 16
LO0 = S - CH


def _maxtree(vals):
    vals = list(vals)
    while len(vals) > 1:
        vals = [
            jnp.maximum(vals[k], vals[k + 1]) for k in range(0, len(vals) - 1, 2)
        ] + ([vals[-1]] if len(vals) % 2 else [])
    return vals[0]


def _sc_body(x_hbm, out_hbm, bufc, bufx, bufy, outa, m_ref, sems, sem2):
    wid = lax.axis_index("s") * 2 + lax.axis_index("c")
    b0 = wid * IPW
    lane = lax.iota(jnp.int32, 16)
    one16 = jnp.ones((16,), jnp.float32)
    zero16f = jnp.zeros((16,), jnp.float32)

    def _copies(i, lo, sem):
        b = b0 + i
        return [
            pltpu.make_async_copy(
                x_hbm.at[pl.ds(lo, CH), ch, buf_i[0]], buf_i[1].at[i], sem
            )
            for ch, buf_i in ((0, (b, bufc)), (1, (b, bufx)), (2, (b, bufy)))
        ]

    def prefetch(i, _):
        for cpy in _copies(i, LO0, sems.at[i]):
            cpy.start()
        return 0

    lax.fori_loop(0, IPW, prefetch, 0)

    def per_item(i, _):
        i_splat = jnp.full((16,), i, jnp.int32)

        def cond(carry):
            c, cmin = carry
            return jnp.logical_or(
                c == 0, jnp.logical_and(c < NCH, cmin == 0)
            )

        def chunk(carry):
            c, _ = carry
            lo = S - CH * (c + 1)
            first = c == 0

            @pl.when(first)
            def _wait0():
                for cpy in _copies(i, LO0, sems.at[i]):
                    cpy.wait()

            @pl.when(jnp.logical_not(first))
            def _fetch_older():
                for cpy in _copies(i, lo, sem2):
                    cpy.start()
                for cpy in _copies(i, lo, sem2):
                    cpy.wait()

            def per_group(g, cmin_acc):
                gl = g * 16
                vlane = gl + lane
                cands = [
                    jnp.where(bufc[i, r, pl.ds(gl, 16)] > 0.0, lo + r + 1, 0)
                    for r in range(CH)
                ]
                mc = _maxtree(cands)
                mo = jnp.where(first, 0, m_ref[pl.ds(gl, 16)])
                newly = jnp.logical_and(mo == 0, mc > 0)
                rrow = jnp.where(newly, mc - 1 - lo, 0)
                x = plsc.load_gather(bufx, [i_splat, rrow, vlane])
                y = plsc.load_gather(bufy, [i_splat, rrow, vlane])
                xo = jnp.where(first, zero16f, outa[1, i, pl.ds(gl, 16)])
                yo = jnp.where(first, zero16f, outa[2, i, pl.ds(gl, 16)])
                outa[1, i, pl.ds(gl, 16)] = jnp.where(newly, x, xo)
                outa[2, i, pl.ds(gl, 16)] = jnp.where(newly, y, yo)
                outa[0, i, pl.ds(gl, 16)] = one16
                mn = jnp.where(mo > 0, mo, mc)
                m_ref[pl.ds(gl, 16)] = mn
                return jnp.minimum(cmin_acc, jnp.min(mn))

            cmin = lax.fori_loop(0, NG, per_group, jnp.int32(2**30))
            return c + 1, cmin

        lax.while_loop(cond, chunk, (0, 0))
        return 0

    lax.fori_loop(0, IPW, per_item, 0)
    pltpu.sync_copy(outa, out_hbm.at[:, pl.ds(b0, IPW)])


def kernel(batch):
    xt = jnp.transpose(batch, (1, 3, 0, 2))
    mesh = plsc.VectorSubcoreMesh(core_axis_name="c", subcore_axis_name="s")
    k = pl.kernel(
        _sc_body,
        out_type=jax.ShapeDtypeStruct((3, B, V), jnp.float32),
        mesh=mesh,
        scratch_types=[
            pltpu.VMEM((IPW, CH, V), jnp.float32),
            pltpu.VMEM((IPW, CH, V), jnp.float32),
            pltpu.VMEM((IPW, CH, V), jnp.float32),
            pltpu.VMEM((3, IPW, V), jnp.float32),
            pltpu.VMEM((V,), jnp.int32),
            pltpu.SemaphoreType.DMA((IPW,)),
            pltpu.SemaphoreType.DMA,
        ],
        compiler_params=pltpu.CompilerParams(
            needs_layout_passes=False, use_tc_tiling_on_sc=False
        ),
    )
    out = k(xt)
    return jnp.transpose(out, (1, 2, 0))
